# baseline (speedup 1.0000x reference)
import functools

import jax
import jax.numpy as jnp
from jax import lax
from jax.experimental import pallas as pl
from jax.experimental.pallas import tpu as pltpu
from jax.experimental.pallas import tpu_sc as plsc

R = 32768
ELEMS = R * 12
NC, NS, L = 2, 16, 16
NW = NC * NS
CHUNK = ELEMS // NW
GROUPS = CHUNK // 48
UNROLL = 4


def _body(x_hbm, out_hbm, x_v, y_v):
    wid = lax.axis_index("s") * NC + lax.axis_index("c")
    base = wid * CHUNK

    pltpu.sync_copy(x_hbm.at[pl.ds(base, CHUNK)], x_v)

    iota = lax.iota(jnp.int32, L)
    pats = []
    for k in range(3):
        f = iota + (16 * k)
        m = lax.rem(f, 12)
        up = jnp.where(jnp.logical_and(m >= 3, m < 6), 3, 0)
        dn = jnp.where(jnp.logical_and(m >= 6, m < 9), -3, 0)
        pats.append(f + up + dn)

    def loop(g, carry):
        b0 = g * (48 * UNROLL)
        for u in range(UNROLL):
            b = b0 + 48 * u
            for k in range(3):
                vec = plsc.load_gather(x_v, [pats[k] + b])
                y_v[pl.ds(b + 16 * k, L)] = vec
        return carry

    lax.fori_loop(0, GROUPS // UNROLL, loop, 0)

    pltpu.sync_copy(y_v, out_hbm.at[pl.ds(base, CHUNK)])


def kernel(X):
    xf = X.reshape(ELEMS)
    mesh = plsc.VectorSubcoreMesh(core_axis_name="c", subcore_axis_name="s")
    run = functools.partial(
        pl.kernel,
        mesh=mesh,
        out_type=jax.ShapeDtypeStruct((ELEMS,), jnp.float32),
        compiler_params=pltpu.CompilerParams(needs_layout_passes=False),
        scratch_types=[
            pltpu.VMEM((CHUNK,), jnp.float32),
            pltpu.VMEM((CHUNK,), jnp.float32),
        ],
    )(_body)
    out = run(xf)
    return out.reshape(R, 4, 3)


# speedup vs baseline: 11.8360x; 11.8360x over previous
import functools

import jax
import jax.numpy as jnp
from jax import lax
from jax.experimental import pallas as pl
from jax.experimental.pallas import tpu as pltpu
from jax.experimental.pallas import tpu_sc as plsc

PERM = (0, 2, 1, 3)
NC, NS = 2, 16
NW = NC * NS
RT = 256
TPW = RT // NW


def _body(x_hbm, out_hbm, x_v, sem):
    wid = lax.axis_index("s") * NC + lax.axis_index("c")
    rt0 = wid * TPW

    copies = []
    for a in range(4):
        copies.append(pltpu.make_async_copy(
            x_hbm.at[:, pl.ds(rt0, TPW), pl.ds(PERM[a], 1)],
            x_v.at[:, :, pl.ds(a, 1)],
            sem,
        ))
    for c in copies:
        c.start()
    for c in copies:
        c.wait()

    pltpu.sync_copy(x_v, out_hbm.at[:, pl.ds(rt0, TPW)])


def kernel(X):
    xv = X.transpose(2, 1, 0).reshape(3, 4, RT, 128).transpose(0, 2, 1, 3)
    mesh = plsc.VectorSubcoreMesh(core_axis_name="c", subcore_axis_name="s")
    run = functools.partial(
        pl.kernel,
        mesh=mesh,
        out_type=jax.ShapeDtypeStruct((3, RT, 4, 128), jnp.float32),
        compiler_params=pltpu.CompilerParams(needs_layout_passes=False),
        scratch_types=[
            pltpu.VMEM((3, TPW, 4, 128), jnp.float32),
            pltpu.SemaphoreType.DMA,
        ],
    )(_body)
    out = run(xv)
    return out.transpose(0, 2, 1, 3).reshape(3, 4, 32768).transpose(2, 1, 0)


# speedup vs baseline: 12.5648x; 1.0616x over previous
import functools

import jax
import jax.numpy as jnp
from jax import lax
from jax.experimental import pallas as pl
from jax.experimental.pallas import tpu as pltpu
from jax.experimental.pallas import tpu_sc as plsc

PERM = (0, 2, 1, 3)
NC, NS = 2, 16
NW = NC * NS
RT = 256
TPW = RT // NW


def _body(x_hbm, out_hbm, x_v, sem):
    wid = lax.axis_index("s") * NC + lax.axis_index("c")
    rt0 = wid * TPW

    pltpu.sync_copy(x_hbm.at[:, pl.ds(rt0, TPW)], x_v)

    def swap_tile(t, carry):
        for c in range(3):
            for k in range(8):
                v1 = x_v[c, t, 1, pl.ds(16 * k, 16)]
                v2 = x_v[c, t, 2, pl.ds(16 * k, 16)]
                x_v[c, t, 1, pl.ds(16 * k, 16)] = v2
                x_v[c, t, 2, pl.ds(16 * k, 16)] = v1
        return carry

    lax.fori_loop(0, TPW, swap_tile, 0)

    pltpu.sync_copy(x_v, out_hbm.at[:, pl.ds(rt0, TPW)])


def kernel(X):
    xv = X.transpose(2, 1, 0).reshape(3, 4, RT, 128).transpose(0, 2, 1, 3)
    mesh = plsc.VectorSubcoreMesh(core_axis_name="c", subcore_axis_name="s")
    run = functools.partial(
        pl.kernel,
        mesh=mesh,
        out_type=jax.ShapeDtypeStruct((3, RT, 4, 128), jnp.float32),
        compiler_params=pltpu.CompilerParams(needs_layout_passes=False),
        scratch_types=[
            pltpu.VMEM((3, TPW, 4, 128), jnp.float32),
            pltpu.SemaphoreType.DMA,
        ],
    )(_body)
    out = run(xv)
    return out.transpose(0, 2, 1, 3).reshape(3, 4, 32768).transpose(2, 1, 0)


# speedup vs baseline: 13.3756x; 1.0645x over previous
import functools

import jax
import jax.numpy as jnp
from jax import lax
from jax.experimental import pallas as pl
from jax.experimental.pallas import tpu as pltpu
from jax.experimental.pallas import tpu_sc as plsc

PERM = (0, 2, 1, 3)
NC = 2
RT = 256
TPW = RT // NC


def _body(x_hbm, out_hbm, x_s, sem):
    cid = lax.axis_index("c")
    rt0 = cid * TPW

    copies = []
    for a in range(4):
        copies.append(pltpu.make_async_copy(
            x_hbm.at[:, pl.ds(rt0, TPW), pl.ds(PERM[a], 1)],
            x_s.at[:, :, pl.ds(a, 1)],
            sem,
        ))
    for c in copies:
        c.start()
    for c in copies:
        c.wait()

    pltpu.sync_copy(x_s, out_hbm.at[:, pl.ds(rt0, TPW)])


def kernel(X):
    xv = X.transpose(2, 1, 0).reshape(3, 4, RT, 128).transpose(0, 2, 1, 3)
    mesh = plsc.ScalarSubcoreMesh(axis_name="c", num_cores=NC)
    run = functools.partial(
        pl.kernel,
        mesh=mesh,
        out_type=jax.ShapeDtypeStruct((3, RT, 4, 128), jnp.float32),
        compiler_params=pltpu.CompilerParams(needs_layout_passes=False),
        scratch_types=[
            pltpu.VMEM_SHARED((3, TPW, 4, 128), jnp.float32),
            pltpu.SemaphoreType.DMA,
        ],
    )(_body)
    out = run(xv)
    return out.transpose(0, 2, 1, 3).reshape(3, 4, 32768).transpose(2, 1, 0)


# speedup vs baseline: 13.3805x; 1.0004x over previous
import functools

import jax
import jax.numpy as jnp
from jax import lax
from jax.experimental import pallas as pl
from jax.experimental.pallas import tpu as pltpu
from jax.experimental.pallas import tpu_sc as plsc

PERM = (0, 2, 1, 3)
NC = 2
RT = 256
TPW = RT // NC


HALF = TPW // 2


def _body(x_hbm, out_hbm, x_s, sem_in0, sem_in1, sem_out):
    cid = lax.axis_index("c")
    rt0 = cid * TPW

    ins = [[], []]
    for h, sem in ((0, sem_in0), (1, sem_in1)):
        for a in range(4):
            ins[h].append(pltpu.make_async_copy(
                x_hbm.at[:, pl.ds(rt0 + h * HALF, HALF), pl.ds(PERM[a], 1)],
                x_s.at[:, pl.ds(h * HALF, HALF), pl.ds(a, 1)],
                sem,
            ))
    outs = [pltpu.make_async_copy(
        x_s.at[:, pl.ds(h * HALF, HALF)],
        out_hbm.at[:, pl.ds(rt0 + h * HALF, HALF)],
        sem_out,
    ) for h in range(2)]

    for h in range(2):
        for c in ins[h]:
            c.start()
    for c in ins[0]:
        c.wait()
    outs[0].start()
    for c in ins[1]:
        c.wait()
    outs[1].start()
    outs[0].wait()
    outs[1].wait()


def kernel(X):
    xv = X.transpose(2, 1, 0).reshape(3, 4, RT, 128).transpose(0, 2, 1, 3)
    mesh = plsc.ScalarSubcoreMesh(axis_name="c", num_cores=NC)
    run = functools.partial(
        pl.kernel,
        mesh=mesh,
        out_type=jax.ShapeDtypeStruct((3, RT, 4, 128), jnp.float32),
        compiler_params=pltpu.CompilerParams(needs_layout_passes=False),
        scratch_types=[
            pltpu.VMEM_SHARED((3, TPW, 4, 128), jnp.float32),
            pltpu.SemaphoreType.DMA,
            pltpu.SemaphoreType.DMA,
            pltpu.SemaphoreType.DMA,
        ],
    )(_body)
    out = run(xv)
    return out.transpose(0, 2, 1, 3).reshape(3, 4, 32768).transpose(2, 1, 0)
